# baseline (speedup 1.0000x reference)
import functools

import jax
import jax.numpy as jnp
from jax import lax
from jax.experimental import pallas as pl
from jax.experimental.pallas import tpu as pltpu
from jax.experimental.pallas import tpu_sc as plsc

N = 10000
E = 320000
D = 128
G = 64

NC = 2
NS = 16
NW = NC * NS
EPW = E // NW
CH = 40
NCHUNK = EPW // CH
DS = 3
IR = 6
RPT = N // NS
ZR = 25
RB = 1000
NB = N // RB


def _softplus_vec(x):
    t = jnp.exp(-jnp.abs(x))
    z = t / (t + 2.0)
    u = z * z
    h = 0.66303605 + u * 0.462632
    h = 2.00005178 + u * h
    return jnp.maximum(x, 0.0) + z * h


@functools.partial(
    pl.kernel,
    mesh=plsc.VectorSubcoreMesh(core_axis_name="c", subcore_axis_name="s"),
    out_type=jax.ShapeDtypeStruct((NC, N, D), jnp.float32),
    scratch_types=[
        pltpu.VMEM((IR, CH), jnp.int32),
        pltpu.VMEM((IR, CH), jnp.int32),
        pltpu.VMEM((DS, CH, D), jnp.float32),
        pltpu.VMEM((DS, CH, D), jnp.float32),
        pltpu.VMEM((ZR, D), jnp.float32),
        pltpu.VMEM_SHARED((N, D), jnp.float32),
    ] + [pltpu.SemaphoreType.DMA] * (3 * DS + IR),
    compiler_params=pltpu.CompilerParams(use_tc_tiling_on_sc=False),
)
def _sc_conv(x_hbm, src4_hbm, dst4_hbm, ea_hbm, out_hbm,
             src_r, dst_r, ea3, gb3, zbuf, acc, *sems):
    c = lax.axis_index("c")
    s = lax.axis_index("s")
    wid = c * NS + s
    sem_e = sems[0:DS]
    sem_g = sems[DS:2 * DS]
    sem_s = sems[2 * DS:3 * DS]
    sem_i = sems[3 * DS:3 * DS + IR]
    ebase = wid * EPW

    def _ids_start(i, isl):
        pltpu.make_async_copy(src4_hbm.at[wid, i], src_r.at[isl],
                              sem_i[isl]).start()
        pltpu.make_async_copy(dst4_hbm.at[wid, i], dst_r.at[isl],
                              sem_i[isl]).start()

    def _ids_wait(i, isl):
        pltpu.make_async_copy(src4_hbm.at[wid, i], src_r.at[isl],
                              sem_i[isl]).wait()
        pltpu.make_async_copy(dst4_hbm.at[wid, i], dst_r.at[isl],
                              sem_i[isl]).wait()

    def _data_start(i, sl, isl):
        pltpu.make_async_copy(ea_hbm.at[pl.ds(ebase + i * CH, CH)],
                              ea3.at[sl], sem_e[sl]).start()
        pltpu.make_async_copy(x_hbm.at[src_r.at[isl]],
                              gb3.at[sl], sem_g[sl]).start()

    def _data_wait(i, sl, isl):
        pltpu.make_async_copy(ea_hbm.at[pl.ds(ebase + i * CH, CH)],
                              ea3.at[sl], sem_e[sl]).wait()
        pltpu.make_async_copy(x_hbm.at[src_r.at[isl]],
                              gb3.at[sl], sem_g[sl]).wait()

    def _scatter_start(sl, isl):
        pltpu.make_async_copy(gb3.at[sl], acc.at[dst_r.at[isl]],
                              sem_s[sl]).start(add=True)

    def _scatter_wait(sl, isl):
        pltpu.make_async_copy(gb3.at[sl], acc.at[dst_r.at[isl]],
                              sem_s[sl]).wait()

    def _compute(sl):
        def _row(r, rc):
            for cc in range(D // 16):
                csl = pl.ds(cc * 16, 16)
                gb3[sl, r, csl] = _softplus_vec(gb3[sl, r, csl]
                                                + ea3[sl, r, csl])
            return rc

        lax.fori_loop(0, CH, _row, 0, unroll=2)

    for i in range(4):
        _ids_start(i, i)

    def _zrow(r, carry):
        for cc in range(D // 16):
            zbuf[r, pl.ds(cc * 16, 16)] = jnp.zeros((16,), jnp.float32)
        return carry

    lax.fori_loop(0, ZR, _zrow, 0)
    for k in range(RPT // ZR):
        pltpu.sync_copy(zbuf, acc.at[pl.ds(s * RPT + k * ZR, ZR)])

    _ids_wait(0, 0)
    _ids_wait(1, 1)
    _data_start(0, 0, 0)
    _data_start(1, 1, 1)
    plsc.subcore_barrier()

    def _block(jj, carry):
        for k in range(6):
            i = 6 * jj + k
            _ids_wait(i + 2, (k + 2) % IR)
            if k == 0:
                @pl.when(jj >= 1)
                def _():
                    _scatter_wait((k - 1) % DS, (k - 1) % IR)
            else:
                _scatter_wait((k - 1) % DS, (k - 1) % IR)
            _data_start(i + 2, (k + 2) % DS, (k + 2) % IR)
            _ids_start(i + 4, (k + 4) % IR)
            _data_wait(i, k % DS, k % IR)
            _compute(k % DS)
            _scatter_start(k % DS, k % IR)
        return carry

    NSTEADY = (NCHUNK - 4) // 6
    lax.fori_loop(0, NSTEADY, _block, 0)

    for i in range(NCHUNK - 4, NCHUNK):
        if i + 2 < NCHUNK:
            _ids_wait(i + 2, (i + 2) % IR)
        _scatter_wait((i - 1) % DS, (i - 1) % IR)
        if i + 2 < NCHUNK:
            _data_start(i + 2, (i + 2) % DS, (i + 2) % IR)
        _data_wait(i, i % DS, i % IR)
        _compute(i % DS)
        _scatter_start(i % DS, i % IR)
    _scatter_wait((NCHUNK - 1) % DS, (NCHUNK - 1) % IR)

    plsc.subcore_barrier()
    pltpu.sync_copy(acc.at[pl.ds(s * RPT, RPT)],
                    out_hbm.at[c, pl.ds(s * RPT, RPT)])


def _mlp_body(p_ref, x_ref, w1_ref, b1_ref, w2_ref, b2_ref, o_ref, *, outer):
    u = p_ref[0] + p_ref[1] + x_ref[...]
    h = jax.nn.softplus(jnp.dot(u, w1_ref[...], preferred_element_type=jnp.float32)
                        + b1_ref[...])
    h = jnp.dot(h, w2_ref[...], preferred_element_type=jnp.float32) + b2_ref[...]
    if outer:
        h = jax.nn.softplus(h)
    o_ref[...] = h


def _mlp_last_body(p_ref, x_ref, w1_ref, b1_ref, w2_ref, b2_ref, bt_ref,
                   gf_ref, nf_ref):
    i = pl.program_id(0)
    u = p_ref[0] + p_ref[1] + x_ref[...]
    h = jax.nn.softplus(jnp.dot(u, w1_ref[...], preferred_element_type=jnp.float32)
                        + b1_ref[...])
    h = jnp.dot(h, w2_ref[...], preferred_element_type=jnp.float32) + b2_ref[...]
    nf_ref[...] = h
    b = bt_ref[0, 0, :]
    onehot = (lax.broadcasted_iota(jnp.int32, (G, RB), 0)
              == b[None, :]).astype(jnp.float32)

    @pl.when(i == 0)
    def _():
        gf_ref[...] = jnp.zeros_like(gf_ref)

    gf_ref[...] += jnp.dot(onehot, h, preferred_element_type=jnp.float32)


def _tc_mid(p, x, w1, b1, w2, b2, outer):
    return pl.pallas_call(
        functools.partial(_mlp_body, outer=outer),
        grid=(NB,),
        in_specs=[
            pl.BlockSpec((NC, RB, D), lambda i: (0, i, 0)),
            pl.BlockSpec((RB, D), lambda i: (i, 0)),
            pl.BlockSpec((D, D), lambda i: (0, 0)),
            pl.BlockSpec((1, D), lambda i: (0, 0)),
            pl.BlockSpec((D, D), lambda i: (0, 0)),
            pl.BlockSpec((1, D), lambda i: (0, 0)),
        ],
        out_specs=pl.BlockSpec((RB, D), lambda i: (i, 0)),
        out_shape=jax.ShapeDtypeStruct((N, D), jnp.float32),
    )(p, x, w1, b1.reshape(1, D), w2, b2.reshape(1, D))


def _tc_last(p, x, w1, b1, w2, b2, batch3):
    return pl.pallas_call(
        _mlp_last_body,
        grid=(NB,),
        in_specs=[
            pl.BlockSpec((NC, RB, D), lambda i: (0, i, 0)),
            pl.BlockSpec((RB, D), lambda i: (i, 0)),
            pl.BlockSpec((D, D), lambda i: (0, 0)),
            pl.BlockSpec((1, D), lambda i: (0, 0)),
            pl.BlockSpec((D, D), lambda i: (0, 0)),
            pl.BlockSpec((1, D), lambda i: (0, 0)),
            pl.BlockSpec((1, 1, RB), lambda i: (i, 0, 0)),
        ],
        out_specs=[
            pl.BlockSpec((G, D), lambda i: (0, 0)),
            pl.BlockSpec((RB, D), lambda i: (i, 0)),
        ],
        out_shape=[
            jax.ShapeDtypeStruct((G, D), jnp.float32),
            jax.ShapeDtypeStruct((N, D), jnp.float32),
        ],
    )(p, x, w1, b1.reshape(1, D), w2, b2.reshape(1, D), batch3)


def kernel(node_attr, edge_index, edge_attr, batch,
           W1_0, b1_0, W2_0, b2_0,
           W1_1, b1_1, W2_1, b2_1,
           W1_2, b1_2, W2_2, b2_2):
    src = edge_index[0].reshape(NW, NCHUNK, CH)
    dst = edge_index[1].reshape(NW, NCHUNK, CH)
    batch3 = batch.reshape(NB, 1, RB)
    params = [(W1_0, b1_0, W2_0, b2_0),
              (W1_1, b1_1, W2_1, b2_1),
              (W1_2, b1_2, W2_2, b2_2)]
    h = node_attr
    for i in range(2):
        partials = _sc_conv(h, src, dst, edge_attr)
        h = _tc_mid(partials, h, *params[i], outer=True)
    partials = _sc_conv(h, src, dst, edge_attr)
    gf, nf = _tc_last(partials, h, *params[2], batch3)
    return gf, nf


# speedup vs baseline: 5.7284x; 5.7284x over previous
import functools

import jax
import jax.numpy as jnp
from jax import lax
from jax.experimental import pallas as pl
from jax.experimental.pallas import tpu as pltpu
from jax.experimental.pallas import tpu_sc as plsc

N = 10000
E = 320000
D = 128
G = 64

NC = 2
NS = 16
NW = NC * NS
EPW = E // NW
CH = 40
NCHUNK = EPW // CH
DS = 3
IR = 6
RPT = N // NS
ZR = 25
RB = 1000
NB = N // RB


def _softplus_vec(x):
    t = jnp.exp(-jnp.abs(x))
    z = t / (t + 2.0)
    u = z * z
    h = 0.66303605 + u * 0.462632
    h = 2.00005178 + u * h
    return jnp.maximum(x, 0.0) + z * h


@functools.partial(
    pl.kernel,
    mesh=plsc.VectorSubcoreMesh(core_axis_name="c", subcore_axis_name="s"),
    out_type=jax.ShapeDtypeStruct((NC, N, D), jnp.float32),
    scratch_types=[
        pltpu.VMEM((IR, CH), jnp.int32),
        pltpu.VMEM((IR, CH), jnp.int32),
        pltpu.VMEM((DS, CH, D), jnp.float32),
        pltpu.VMEM((DS, CH, D), jnp.float32),
        pltpu.VMEM((ZR, D), jnp.float32),
        pltpu.VMEM_SHARED((N, D), jnp.float32),
    ] + [pltpu.SemaphoreType.DMA] * (3 * DS + IR),
    compiler_params=pltpu.CompilerParams(use_tc_tiling_on_sc=False),
)
def _sc_conv(x_hbm, src4_hbm, dst4_hbm, ea_hbm, out_hbm,
             src_r, dst_r, ea3, gb3, zbuf, acc, *sems):
    c = lax.axis_index("c")
    s = lax.axis_index("s")
    wid = c * NS + s
    sem_e = sems[0:DS]
    sem_g = sems[DS:2 * DS]
    sem_s = sems[2 * DS:3 * DS]
    sem_i = sems[3 * DS:3 * DS + IR]
    ebase = wid * EPW

    def _ids_start(i, isl):
        pltpu.make_async_copy(src4_hbm.at[wid, i], src_r.at[isl],
                              sem_i[isl]).start()
        pltpu.make_async_copy(dst4_hbm.at[wid, i], dst_r.at[isl],
                              sem_i[isl]).start()

    def _ids_wait(i, isl):
        pltpu.make_async_copy(src4_hbm.at[wid, i], src_r.at[isl],
                              sem_i[isl]).wait()
        pltpu.make_async_copy(dst4_hbm.at[wid, i], dst_r.at[isl],
                              sem_i[isl]).wait()

    def _data_start(i, sl, isl):
        pltpu.make_async_copy(ea_hbm.at[pl.ds(ebase + i * CH, CH)],
                              ea3.at[sl], sem_e[sl]).start()
        pltpu.make_async_copy(x_hbm.at[src_r.at[isl]],
                              gb3.at[sl], sem_g[sl]).start()

    def _data_wait(i, sl, isl):
        pltpu.make_async_copy(ea_hbm.at[pl.ds(ebase + i * CH, CH)],
                              ea3.at[sl], sem_e[sl]).wait()
        pltpu.make_async_copy(x_hbm.at[src_r.at[isl]],
                              gb3.at[sl], sem_g[sl]).wait()

    def _scatter_start(sl, isl):
        pltpu.make_async_copy(gb3.at[sl], acc.at[dst_r.at[isl]],
                              sem_s[sl]).start(add=True)

    def _scatter_wait(sl, isl):
        pltpu.make_async_copy(gb3.at[sl], acc.at[dst_r.at[isl]],
                              sem_s[sl]).wait()

    def _compute(sl):
        def _row(r, rc):
            for cc in range(D // 16):
                csl = pl.ds(cc * 16, 16)
                gb3[sl, r, csl] = _softplus_vec(gb3[sl, r, csl]
                                                + ea3[sl, r, csl])
            return rc

        lax.fori_loop(0, CH, _row, 0)

    for i in range(4):
        _ids_start(i, i)

    def _zrow(r, carry):
        for cc in range(D // 16):
            zbuf[r, pl.ds(cc * 16, 16)] = jnp.zeros((16,), jnp.float32)
        return carry

    lax.fori_loop(0, ZR, _zrow, 0)
    for k in range(RPT // ZR):
        pltpu.sync_copy(zbuf, acc.at[pl.ds(s * RPT + k * ZR, ZR)])

    _ids_wait(0, 0)
    _ids_wait(1, 1)
    _data_start(0, 0, 0)
    _data_start(1, 1, 1)
    plsc.subcore_barrier()

    def _block(jj, carry):
        for k in range(6):
            i = 6 * jj + k
            _ids_wait(i + 2, (k + 2) % IR)
            if k == 0:
                @pl.when(jj >= 1)
                def _():
                    _scatter_wait((k - 1) % DS, (k - 1) % IR)
            else:
                _scatter_wait((k - 1) % DS, (k - 1) % IR)
            _data_start(i + 2, (k + 2) % DS, (k + 2) % IR)
            _ids_start(i + 4, (k + 4) % IR)
            _data_wait(i, k % DS, k % IR)
            _compute(k % DS)
            _scatter_start(k % DS, k % IR)
        return carry

    NSTEADY = (NCHUNK - 4) // 6
    lax.fori_loop(0, NSTEADY, _block, 0)

    for i in range(NCHUNK - 4, NCHUNK):
        if i + 2 < NCHUNK:
            _ids_wait(i + 2, (i + 2) % IR)
        _scatter_wait((i - 1) % DS, (i - 1) % IR)
        if i + 2 < NCHUNK:
            _data_start(i + 2, (i + 2) % DS, (i + 2) % IR)
        _data_wait(i, i % DS, i % IR)
        _compute(i % DS)
        _scatter_start(i % DS, i % IR)
    _scatter_wait((NCHUNK - 1) % DS, (NCHUNK - 1) % IR)

    plsc.subcore_barrier()
    pltpu.sync_copy(acc.at[pl.ds(s * RPT, RPT)],
                    out_hbm.at[c, pl.ds(s * RPT, RPT)])


def _mlp_body(p_ref, x_ref, w1_ref, b1_ref, w2_ref, b2_ref, o_ref, *, outer):
    u = p_ref[0] + p_ref[1] + x_ref[...]
    h = jax.nn.softplus(jnp.dot(u, w1_ref[...], preferred_element_type=jnp.float32)
                        + b1_ref[...])
    h = jnp.dot(h, w2_ref[...], preferred_element_type=jnp.float32) + b2_ref[...]
    if outer:
        h = jax.nn.softplus(h)
    o_ref[...] = h


def _mlp_last_body(p_ref, x_ref, w1_ref, b1_ref, w2_ref, b2_ref, bt_ref,
                   gf_ref, nf_ref):
    i = pl.program_id(0)
    u = p_ref[0] + p_ref[1] + x_ref[...]
    h = jax.nn.softplus(jnp.dot(u, w1_ref[...], preferred_element_type=jnp.float32)
                        + b1_ref[...])
    h = jnp.dot(h, w2_ref[...], preferred_element_type=jnp.float32) + b2_ref[...]
    nf_ref[...] = h
    b = bt_ref[0, 0, :]
    onehot = (lax.broadcasted_iota(jnp.int32, (G, RB), 0)
              == b[None, :]).astype(jnp.float32)

    @pl.when(i == 0)
    def _():
        gf_ref[...] = jnp.zeros_like(gf_ref)

    gf_ref[...] += jnp.dot(onehot, h, preferred_element_type=jnp.float32)


def _tc_mid(p, x, w1, b1, w2, b2, outer):
    return pl.pallas_call(
        functools.partial(_mlp_body, outer=outer),
        grid=(NB,),
        in_specs=[
            pl.BlockSpec((NC, RB, D), lambda i: (0, i, 0)),
            pl.BlockSpec((RB, D), lambda i: (i, 0)),
            pl.BlockSpec((D, D), lambda i: (0, 0)),
            pl.BlockSpec((1, D), lambda i: (0, 0)),
            pl.BlockSpec((D, D), lambda i: (0, 0)),
            pl.BlockSpec((1, D), lambda i: (0, 0)),
        ],
        out_specs=pl.BlockSpec((RB, D), lambda i: (i, 0)),
        out_shape=jax.ShapeDtypeStruct((N, D), jnp.float32),
    )(p, x, w1, b1.reshape(1, D), w2, b2.reshape(1, D))


def _tc_last(p, x, w1, b1, w2, b2, batch3):
    return pl.pallas_call(
        _mlp_last_body,
        grid=(NB,),
        in_specs=[
            pl.BlockSpec((NC, RB, D), lambda i: (0, i, 0)),
            pl.BlockSpec((RB, D), lambda i: (i, 0)),
            pl.BlockSpec((D, D), lambda i: (0, 0)),
            pl.BlockSpec((1, D), lambda i: (0, 0)),
            pl.BlockSpec((D, D), lambda i: (0, 0)),
            pl.BlockSpec((1, D), lambda i: (0, 0)),
            pl.BlockSpec((1, 1, RB), lambda i: (i, 0, 0)),
        ],
        out_specs=[
            pl.BlockSpec((G, D), lambda i: (0, 0)),
            pl.BlockSpec((RB, D), lambda i: (i, 0)),
        ],
        out_shape=[
            jax.ShapeDtypeStruct((G, D), jnp.float32),
            jax.ShapeDtypeStruct((N, D), jnp.float32),
        ],
    )(p, x, w1, b1.reshape(1, D), w2, b2.reshape(1, D), batch3)


def kernel(node_attr, edge_index, edge_attr, batch,
           W1_0, b1_0, W2_0, b2_0,
           W1_1, b1_1, W2_1, b2_1,
           W1_2, b1_2, W2_2, b2_2):
    src = edge_index[0].reshape(NW, NCHUNK, CH)
    dst = edge_index[1].reshape(NW, NCHUNK, CH)
    batch3 = batch.reshape(NB, 1, RB)
    params = [(W1_0, b1_0, W2_0, b2_0),
              (W1_1, b1_1, W2_1, b2_1),
              (W1_2, b1_2, W2_2, b2_2)]
    h = node_attr
    for i in range(2):
        partials = _sc_conv(h, src, dst, edge_attr)
        h = _tc_mid(partials, h, *params[i], outer=True)
    partials = _sc_conv(h, src, dst, edge_attr)
    gf, nf = _tc_last(partials, h, *params[2], batch3)
    return gf, nf
